# baseline (speedup 1.0000x reference)
import functools

import jax
import jax.numpy as jnp
from jax import lax
from jax.experimental import pallas as pl
from jax.experimental.pallas import tpu as pltpu
from jax.experimental.pallas import tpu_sc as plsc

N = 10000
E = 320000
D = 128
HALF = 64
HOP = 10

NS = 16
NC = 2
C = 128
CHUNKS = 160
EPT = CHUNKS * C
NPT = N // NS


def _mlp1_body(x_ref, w1t_ref, b1_ref, w2t_ref, b2_ref, a2_ref, x2s_ref, rs_ref):
    h = jnp.dot(x_ref[...], w1t_ref[...], preferred_element_type=jnp.float32)
    h = jnp.maximum(h + b1_ref[...], 0.0)
    x2 = jnp.dot(h, w2t_ref[...], preferred_element_type=jnp.float32) + b2_ref[...]
    r = a2_ref[...] * x2
    x2s_ref[0] = x2[:, :HALF]
    x2s_ref[1] = x2[:, HALF:]
    rs_ref[0] = r[:, :HALF]
    rs_ref[1] = r[:, HALF:]


def _mlp2_body(hs_ref, w3t_ref, b3_ref, out_ref):
    h = jnp.concatenate([hs_ref[0], hs_ref[1]], axis=-1)
    h = jnp.maximum(h, 0.0)
    out_ref[...] = jnp.dot(h, w3t_ref[...], preferred_element_type=jnp.float32) + b3_ref[...]


def _sc_body(x2_hbm, r_hbm, e3_hbm, out_hbm, buf_a, buf_b,
             eb0, eb1, eb2, eb3, rows0, rows1, rows2, rows3,
             sidx0, sidx1, sidx2, sidx3,
             esem0, esem1, esem2, esem3,
             gsem0, gsem1, gsem2, gsem3, ssem0, ssem1, ssem2, ssem3):
    cid = lax.axis_index("c")
    sid = lax.axis_index("s")
    node_lo = sid * NPT
    ebs = [eb0, eb1, eb2, eb3]
    esems = [esem0, esem1, esem2, esem3]
    rows = [rows0, rows1, rows2, rows3]
    sidx = [sidx0, sidx1, sidx2, sidx3]
    gsems = [gsem0, gsem1, gsem2, gsem3]
    ssems = [ssem0, ssem1, ssem2, ssem3]

    pltpu.sync_copy(x2_hbm.at[cid, pl.ds(node_lo, NPT)],
                    buf_a.at[pl.ds(node_lo, NPT)])

    def scale(eb, rows_v, sidx_v):
        for g in range(C // 16):
            sidx_v[pl.ds(g * 16, 16)] = eb[1, pl.ds(g * 16, 16)]

        @pl.loop(0, C, unroll=16)
        def _edge(e):
            wi = plsc.load_gather(
                eb, [jnp.full((16,), 2, jnp.int32), jnp.full((16,), e, jnp.int32)])
            wv = plsc.bitcast(wi, jnp.float32)
            for d in range(HALF // 16):
                sl = pl.ds(d * 16, 16)
                rows_v[e, sl] = rows_v[e, sl] * wv

    def hop(src, dst):
        pltpu.sync_copy(r_hbm.at[cid, pl.ds(node_lo, NPT)],
                        dst.at[pl.ds(node_lo, NPT)])
        plsc.subcore_barrier()

        for p in range(3):
            pltpu.async_copy(e3_hbm.at[sid, p], ebs[p], esems[p])
        pltpu.make_async_copy(e3_hbm.at[sid, 0], ebs[0], esems[0]).wait()
        pltpu.async_copy(src.at[ebs[0].at[0]], rows[0], gsems[0])

        @pl.loop(0, CHUNKS // 4)
        def _quad(jj):
            base = 4 * jj
            for p in range(4):
                q = base + p
                pn = (p + 1) % 4
                pf = (p + 3) % 4

                @pl.when(q + 1 < CHUNKS)
                def _():
                    pltpu.make_async_copy(
                        e3_hbm.at[sid, 0], ebs[pn], esems[pn]).wait()

                    @pl.when(q >= 3)
                    def _():
                        pltpu.make_async_copy(
                            rows[pn], dst.at[sidx[pn]], ssems[pn]).wait()

                    pltpu.async_copy(src.at[ebs[pn].at[0]], rows[pn], gsems[pn])

                    @pl.when(q + 3 < CHUNKS)
                    def _():
                        pltpu.async_copy(
                            e3_hbm.at[sid, q + 3], ebs[pf], esems[pf])

                pltpu.make_async_copy(
                    src.at[ebs[p].at[0]], rows[p], gsems[p]).wait()
                scale(ebs[p], rows[p], sidx[p])
                pltpu.async_copy(rows[p], dst.at[sidx[p]], ssems[p],
                                 add=True)

        for i in (0, 1, 2, 3):
            pltpu.make_async_copy(rows[i], dst.at[sidx[i]], ssems[i]).wait()
        plsc.subcore_barrier()

    @pl.loop(0, HOP // 2)
    def _hop_pair(_):
        hop(buf_a, buf_b)
        hop(buf_b, buf_a)

    pltpu.sync_copy(buf_a.at[pl.ds(node_lo, NPT)],
                    out_hbm.at[cid, pl.ds(node_lo, NPT)])


@jax.jit
def kernel(x, edge_weight, A2, W1, b1, W2, b2, W3, b3, edge_index):
    row = edge_index[0]
    col = edge_index[1]
    pad = NS * EPT - E
    row_p = jnp.concatenate([row, jnp.zeros((pad,), jnp.int32)]).reshape(NS, CHUNKS, 1, C)
    col_p = jnp.concatenate([col, jnp.zeros((pad,), jnp.int32)]).reshape(NS, CHUNKS, 1, C)
    w_p = jax.lax.bitcast_convert_type(
        jnp.concatenate([edge_weight, jnp.zeros((pad,), jnp.float32)]), jnp.int32
    ).reshape(NS, CHUNKS, 1, C)
    e3 = jnp.concatenate([col_p, row_p, w_p], axis=2)

    x2s, rs = pl.pallas_call(
        _mlp1_body,
        out_shape=(
            jax.ShapeDtypeStruct((NC, N, HALF), jnp.float32),
            jax.ShapeDtypeStruct((NC, N, HALF), jnp.float32),
        ),
    )(x, W1.T, b1.reshape(1, D), W2.T, b2.reshape(1, D), A2)

    sc_fn = pl.kernel(
        _sc_body,
        out_type=jax.ShapeDtypeStruct((NC, N, HALF), jnp.float32),
        mesh=plsc.VectorSubcoreMesh(core_axis_name="c", subcore_axis_name="s"),
        compiler_params=pltpu.CompilerParams(
            use_tc_tiling_on_sc=False, needs_layout_passes=False),
        scratch_types=[
            pltpu.VMEM_SHARED((N, HALF), jnp.float32),
            pltpu.VMEM_SHARED((N, HALF), jnp.float32),
            pltpu.VMEM((3, C), jnp.int32),
            pltpu.VMEM((3, C), jnp.int32),
            pltpu.VMEM((3, C), jnp.int32),
            pltpu.VMEM((3, C), jnp.int32),
            pltpu.VMEM((C, HALF), jnp.float32),
            pltpu.VMEM((C, HALF), jnp.float32),
            pltpu.VMEM((C, HALF), jnp.float32),
            pltpu.VMEM((C, HALF), jnp.float32),
            pltpu.VMEM((C,), jnp.int32),
            pltpu.VMEM((C,), jnp.int32),
            pltpu.VMEM((C,), jnp.int32),
            pltpu.VMEM((C,), jnp.int32),
            pltpu.SemaphoreType.DMA,
            pltpu.SemaphoreType.DMA,
            pltpu.SemaphoreType.DMA,
            pltpu.SemaphoreType.DMA,
            pltpu.SemaphoreType.DMA,
            pltpu.SemaphoreType.DMA,
            pltpu.SemaphoreType.DMA,
            pltpu.SemaphoreType.DMA,
            pltpu.SemaphoreType.DMA,
            pltpu.SemaphoreType.DMA,
            pltpu.SemaphoreType.DMA,
            pltpu.SemaphoreType.DMA,
        ],
    )
    hs = sc_fn(x2s, rs, e3)

    out = pl.pallas_call(
        _mlp2_body,
        out_shape=jax.ShapeDtypeStruct((N, D), jnp.float32),
    )(hs, W3.T, b3.reshape(1, D))
    return out


# speedup vs baseline: 2.3966x; 2.3966x over previous
import functools

import jax
import jax.numpy as jnp
from jax import lax
from jax.experimental import pallas as pl
from jax.experimental.pallas import tpu as pltpu
from jax.experimental.pallas import tpu_sc as plsc

N = 10000
E = 320000
D = 128
HALF = 64
HOP = 10

NS = 16
NC = 2
C = 128
CHUNKS = 160
EPT = CHUNKS * C
NPT = N // NS


def _mlp1_body(x_ref, w1t_ref, b1_ref, w2t_ref, b2_ref, a2_ref, x2s_ref, rs_ref):
    h = jnp.dot(x_ref[...], w1t_ref[...], preferred_element_type=jnp.float32)
    h = jnp.maximum(h + b1_ref[...], 0.0)
    x2 = jnp.dot(h, w2t_ref[...], preferred_element_type=jnp.float32) + b2_ref[...]
    r = a2_ref[...] * x2
    x2s_ref[0] = x2[:, :HALF]
    x2s_ref[1] = x2[:, HALF:]
    rs_ref[0] = r[:, :HALF]
    rs_ref[1] = r[:, HALF:]


def _mlp2_body(hs_ref, w3t_ref, b3_ref, out_ref):
    h = jnp.concatenate([hs_ref[0], hs_ref[1]], axis=-1)
    h = jnp.maximum(h, 0.0)
    out_ref[...] = jnp.dot(h, w3t_ref[...], preferred_element_type=jnp.float32) + b3_ref[...]


def _sc_body(x2_hbm, r_hbm, e3_hbm, out_hbm, buf_a, buf_b,
             eb0, eb1, eb2, eb3, rows0, rows1, rows2, rows3,
             sidx0, sidx1, sidx2, sidx3,
             esem0, esem1, esem2, esem3,
             gsem0, gsem1, gsem2, gsem3, ssem0, ssem1, ssem2, ssem3):
    cid = lax.axis_index("c")
    sid = lax.axis_index("s")
    node_lo = sid * NPT
    ebs = [eb0, eb1, eb2, eb3]
    esems = [esem0, esem1, esem2, esem3]
    rows = [rows0, rows1, rows2, rows3]
    sidx = [sidx0, sidx1, sidx2, sidx3]
    gsems = [gsem0, gsem1, gsem2, gsem3]
    ssems = [ssem0, ssem1, ssem2, ssem3]

    pltpu.sync_copy(x2_hbm.at[cid, pl.ds(node_lo, NPT)],
                    buf_a.at[pl.ds(node_lo, NPT)])

    def scale(eb, rows_v, sidx_v):
        for g in range(C // 16):
            sidx_v[pl.ds(g * 16, 16)] = eb[1, pl.ds(g * 16, 16)]

        @pl.loop(0, C, unroll=4)
        def _edge(e):
            wi = plsc.load_gather(
                eb, [jnp.full((16,), 2, jnp.int32), jnp.full((16,), e, jnp.int32)])
            wv = plsc.bitcast(wi, jnp.float32)
            for d in range(HALF // 16):
                sl = pl.ds(d * 16, 16)
                rows_v[e, sl] = rows_v[e, sl] * wv

    def hop(src, dst):
        pltpu.sync_copy(r_hbm.at[cid, pl.ds(node_lo, NPT)],
                        dst.at[pl.ds(node_lo, NPT)])
        plsc.subcore_barrier()

        for p in range(3):
            pltpu.async_copy(e3_hbm.at[sid, p], ebs[p], esems[p])
        pltpu.make_async_copy(e3_hbm.at[sid, 0], ebs[0], esems[0]).wait()
        pltpu.async_copy(src.at[ebs[0].at[0]], rows[0], gsems[0])

        @pl.loop(0, CHUNKS // 4)
        def _quad(jj):
            base = 4 * jj
            for p in range(4):
                q = base + p
                pn = (p + 1) % 4
                pf = (p + 3) % 4

                @pl.when(q + 1 < CHUNKS)
                def _():
                    pltpu.make_async_copy(
                        e3_hbm.at[sid, 0], ebs[pn], esems[pn]).wait()

                    @pl.when(q >= 3)
                    def _():
                        pltpu.make_async_copy(
                            rows[pn], dst.at[sidx[pn]], ssems[pn]).wait()

                    pltpu.async_copy(src.at[ebs[pn].at[0]], rows[pn], gsems[pn])

                    @pl.when(q + 3 < CHUNKS)
                    def _():
                        pltpu.async_copy(
                            e3_hbm.at[sid, q + 3], ebs[pf], esems[pf])

                pltpu.make_async_copy(
                    src.at[ebs[p].at[0]], rows[p], gsems[p]).wait()
                scale(ebs[p], rows[p], sidx[p])
                pltpu.async_copy(rows[p], dst.at[sidx[p]], ssems[p],
                                 add=True)

        for i in (0, 1, 2, 3):
            pltpu.make_async_copy(rows[i], dst.at[sidx[i]], ssems[i]).wait()
        plsc.subcore_barrier()

    @pl.loop(0, HOP // 2)
    def _hop_pair(_):
        hop(buf_a, buf_b)
        hop(buf_b, buf_a)

    pltpu.sync_copy(buf_a.at[pl.ds(node_lo, NPT)],
                    out_hbm.at[cid, pl.ds(node_lo, NPT)])


@jax.jit
def kernel(x, edge_weight, A2, W1, b1, W2, b2, W3, b3, edge_index):
    row = edge_index[0]
    col = edge_index[1]
    pad = NS * EPT - E
    row_p = jnp.concatenate([row, jnp.zeros((pad,), jnp.int32)]).reshape(NS, CHUNKS, 1, C)
    col_p = jnp.concatenate([col, jnp.zeros((pad,), jnp.int32)]).reshape(NS, CHUNKS, 1, C)
    w_p = jax.lax.bitcast_convert_type(
        jnp.concatenate([edge_weight, jnp.zeros((pad,), jnp.float32)]), jnp.int32
    ).reshape(NS, CHUNKS, 1, C)
    e3 = jnp.concatenate([col_p, row_p, w_p], axis=2)

    x2s, rs = pl.pallas_call(
        _mlp1_body,
        out_shape=(
            jax.ShapeDtypeStruct((NC, N, HALF), jnp.float32),
            jax.ShapeDtypeStruct((NC, N, HALF), jnp.float32),
        ),
    )(x, W1.T, b1.reshape(1, D), W2.T, b2.reshape(1, D), A2)

    sc_fn = pl.kernel(
        _sc_body,
        out_type=jax.ShapeDtypeStruct((NC, N, HALF), jnp.float32),
        mesh=plsc.VectorSubcoreMesh(core_axis_name="c", subcore_axis_name="s"),
        compiler_params=pltpu.CompilerParams(
            use_tc_tiling_on_sc=False, needs_layout_passes=False),
        scratch_types=[
            pltpu.VMEM_SHARED((N, HALF), jnp.float32),
            pltpu.VMEM_SHARED((N, HALF), jnp.float32),
            pltpu.VMEM((3, C), jnp.int32),
            pltpu.VMEM((3, C), jnp.int32),
            pltpu.VMEM((3, C), jnp.int32),
            pltpu.VMEM((3, C), jnp.int32),
            pltpu.VMEM((C, HALF), jnp.float32),
            pltpu.VMEM((C, HALF), jnp.float32),
            pltpu.VMEM((C, HALF), jnp.float32),
            pltpu.VMEM((C, HALF), jnp.float32),
            pltpu.VMEM((C,), jnp.int32),
            pltpu.VMEM((C,), jnp.int32),
            pltpu.VMEM((C,), jnp.int32),
            pltpu.VMEM((C,), jnp.int32),
            pltpu.SemaphoreType.DMA,
            pltpu.SemaphoreType.DMA,
            pltpu.SemaphoreType.DMA,
            pltpu.SemaphoreType.DMA,
            pltpu.SemaphoreType.DMA,
            pltpu.SemaphoreType.DMA,
            pltpu.SemaphoreType.DMA,
            pltpu.SemaphoreType.DMA,
            pltpu.SemaphoreType.DMA,
            pltpu.SemaphoreType.DMA,
            pltpu.SemaphoreType.DMA,
            pltpu.SemaphoreType.DMA,
        ],
    )
    hs = sc_fn(x2s, rs, e3)

    out = pl.pallas_call(
        _mlp2_body,
        out_shape=jax.ShapeDtypeStruct((N, D), jnp.float32),
    )(hs, W3.T, b3.reshape(1, D))
    return out
